# baseline (speedup 1.0000x reference)
import functools

import jax
import jax.numpy as jnp
from jax import lax
from jax.experimental import pallas as pl
from jax.experimental.pallas import tpu as pltpu
from jax.experimental.pallas import tpu_sc as plsc

_B = 16384
_C = 1000
_NC = 2
_NS = 16
_L = 16
_NW = _NC * _NS
_RPW = _B // _NW
_WROWS = 32
_NWIN = _RPW // _WROWS

_mesh = plsc.VectorSubcoreMesh(core_axis_name="c", subcore_axis_name="s")


@functools.partial(
    pl.kernel,
    mesh=_mesh,
    out_type=jax.ShapeDtypeStruct((_NW, _L), jnp.float32),
    scratch_types=[
        pltpu.VMEM((_RPW,), jnp.int32),
        pltpu.VMEM((_WROWS, _C), jnp.float32),
        pltpu.VMEM((_L,), jnp.float32),
    ],
    compiler_params=pltpu.CompilerParams(use_tc_tiling_on_sc=True,
                                         needs_layout_passes=False),
)
def _stream_extract(pred_hbm, lab_hbm, out_hbm, lab_v, win_v, acc_v):
    wid = lax.axis_index("s") * _NC + lax.axis_index("c")
    base = wid * _RPW
    pltpu.sync_copy(lab_hbm.at[pl.ds(base, _RPW)], lab_v)

    iota = lax.iota(jnp.int32, _L)
    acc = jnp.zeros((_L,), jnp.float32)
    for t in range(_NWIN):
        pltpu.sync_copy(pred_hbm.at[pl.ds(base + t * _WROWS, _WROWS), :],
                        win_v)
        for j in range(_WROWS // _L):
            cols = lab_v[pl.ds(t * _WROWS + j * _L, _L)]
            rows = iota + j * _L
            acc = acc + plsc.load_gather(win_v, [rows, cols])
    acc_v[...] = acc
    pltpu.sync_copy(acc_v, out_hbm.at[wid])


def kernel(predict, label):
    partial = _stream_extract(predict, label.astype(jnp.int32))
    return partial.sum() / predict.shape[0]


# speedup vs baseline: 1.0515x; 1.0515x over previous
import functools

import jax
import jax.numpy as jnp
from jax import lax
from jax.experimental import pallas as pl
from jax.experimental.pallas import tpu as pltpu
from jax.experimental.pallas import tpu_sc as plsc

_B = 16384
_C = 1000
_NC = 2
_NS = 16
_L = 16
_NW = _NC * _NS
_RPW = _B // _NW
_WROWS = 32
_NWIN = _RPW // _WROWS

_mesh = plsc.VectorSubcoreMesh(core_axis_name="c", subcore_axis_name="s")


@functools.partial(
    pl.kernel,
    mesh=_mesh,
    out_type=jax.ShapeDtypeStruct((_NW, _L), jnp.float32),
    scratch_types=[
        pltpu.VMEM((_RPW,), jnp.int32),
        pltpu.VMEM((_WROWS, _C), jnp.float32),
        pltpu.VMEM((_WROWS, _C), jnp.float32),
        pltpu.VMEM((_L,), jnp.float32),
        pltpu.SemaphoreType.DMA,
        pltpu.SemaphoreType.DMA,
    ],
    compiler_params=pltpu.CompilerParams(use_tc_tiling_on_sc=True,
                                         needs_layout_passes=False),
)
def _stream_extract(pred_hbm, lab_hbm, out_hbm, lab_v, win_a, win_b, acc_v,
                    sem_a, sem_b):
    wid = lax.axis_index("s") * _NC + lax.axis_index("c")
    base = wid * _RPW
    pltpu.sync_copy(lab_hbm.at[pl.ds(base, _RPW)], lab_v)

    bufs = (win_a, win_b)
    sems = (sem_a, sem_b)

    def issue(t):
        return pltpu.async_copy(
            pred_hbm.at[pl.ds(base + t * _WROWS, _WROWS), :],
            bufs[t % 2], sems[t % 2])

    iota = lax.iota(jnp.int32, _L)
    acc = jnp.zeros((_L,), jnp.float32)
    copies = [issue(0), None]
    for t in range(_NWIN):
        if t + 1 < _NWIN:
            copies[(t + 1) % 2] = issue(t + 1)
        copies[t % 2].wait()
        win = bufs[t % 2]
        for j in range(_WROWS // _L):
            cols = lab_v[pl.ds(t * _WROWS + j * _L, _L)]
            rows = iota + j * _L
            acc = acc + plsc.load_gather(win, [rows, cols])
    acc_v[...] = acc
    pltpu.sync_copy(acc_v, out_hbm.at[wid])


def kernel(predict, label):
    partial = _stream_extract(predict, label.astype(jnp.int32))
    return partial.sum() / predict.shape[0]


# speedup vs baseline: 1.0566x; 1.0049x over previous
import functools

import jax
import jax.numpy as jnp
from jax import lax
from jax.experimental import pallas as pl
from jax.experimental.pallas import tpu as pltpu
from jax.experimental.pallas import tpu_sc as plsc

_B = 16384
_C = 1000
_NC = 2
_NS = 16
_L = 16
_NW = _NC * _NS
_RPW = _B // _NW
_WROWS = 16
_NWIN = _RPW // _WROWS
_NBUF = 4

_mesh = plsc.VectorSubcoreMesh(core_axis_name="c", subcore_axis_name="s")


@functools.partial(
    pl.kernel,
    mesh=_mesh,
    out_type=jax.ShapeDtypeStruct((_NW, _L), jnp.float32),
    scratch_types=[
        pltpu.VMEM((_RPW,), jnp.int32),
        [pltpu.VMEM((_WROWS, _C), jnp.float32) for _ in range(_NBUF)],
        pltpu.VMEM((_L,), jnp.float32),
        [pltpu.SemaphoreType.DMA for _ in range(_NBUF)],
    ],
    compiler_params=pltpu.CompilerParams(use_tc_tiling_on_sc=True,
                                         needs_layout_passes=False),
)
def _stream_extract(pred_hbm, lab_hbm, out_hbm, lab_v, bufs, acc_v, sems):
    wid = lax.axis_index("s") * _NC + lax.axis_index("c")
    base = wid * _RPW
    pltpu.sync_copy(lab_hbm.at[pl.ds(base, _RPW)], lab_v)

    def issue(t):
        return pltpu.async_copy(
            pred_hbm.at[pl.ds(base + t * _WROWS, _WROWS), :],
            bufs[t % _NBUF], sems[t % _NBUF])

    iota = lax.iota(jnp.int32, _L)
    acc = jnp.zeros((_L,), jnp.float32)
    copies = [None] * _NBUF
    for t in range(min(_NBUF, _NWIN)):
        copies[t] = issue(t)
    for t in range(_NWIN):
        copies[t % _NBUF].wait()
        win = bufs[t % _NBUF]
        for j in range(_WROWS // _L):
            cols = lab_v[pl.ds(t * _WROWS + j * _L, _L)]
            rows = iota + j * _L
            acc = acc + plsc.load_gather(win, [rows, cols])
        nxt = t + _NBUF
        if nxt < _NWIN:
            copies[t % _NBUF] = issue(nxt)
    acc_v[...] = acc
    pltpu.sync_copy(acc_v, out_hbm.at[wid])


def kernel(predict, label):
    partial = _stream_extract(predict, label.astype(jnp.int32))
    return partial.sum() / predict.shape[0]


# speedup vs baseline: 2.8048x; 2.6544x over previous
import functools

import jax
import jax.numpy as jnp
from jax import lax
from jax.experimental import pallas as pl
from jax.experimental.pallas import tpu as pltpu
from jax.experimental.pallas import tpu_sc as plsc

_B = 16384
_C = 1000
_BLK = 512
_NBLK = _B // _BLK


def _tc_body(lab_ref, pred_ref, out_ref, acc_ref):
    i = pl.program_id(0)
    lab = lab_ref[0, 0, :]
    rows = lax.broadcasted_iota(jnp.int32, (_C, _BLK), 0)
    sel = rows == lab[None, :]
    part = jnp.sum(jnp.where(sel, pred_ref[...], 0.0))

    @pl.when(i == 0)
    def _():
        acc_ref[0] = 0.0

    acc_ref[0] += part

    @pl.when(i == _NBLK - 1)
    def _():
        out_ref[0, 0] = acc_ref[0]


_tc_call = pl.pallas_call(
    _tc_body,
    grid=(_NBLK,),
    in_specs=[
        pl.BlockSpec((1, 1, _BLK), lambda i: (i, 0, 0)),
        pl.BlockSpec((_C, _BLK), lambda i: (0, i)),
    ],
    out_specs=pl.BlockSpec(memory_space=pltpu.SMEM),
    out_shape=jax.ShapeDtypeStruct((1, 1), jnp.float32),
    scratch_shapes=[pltpu.SMEM((1,), jnp.float32)],
)


def kernel(predict, label):
    lab3 = label.astype(jnp.int32).reshape(_NBLK, 1, _BLK)
    total = _tc_call(lab3, predict.T)
    return total[0, 0] / predict.shape[0]
